# baseline (speedup 1.0000x reference)
import numpy as np
import jax
import jax.numpy as jnp
from jax.experimental import pallas as pl
from jax.experimental.pallas import tpu as pltpu

FMAP = 26
IMG = 224
DF = 1536
KLIB = 16384
Q = FMAP * FMAP

BK = 2048
NB = KLIB // BK
BK2 = 4096
NB2 = KLIB // BK2

_INT_MAX = np.int32(2**31 - 1)


def _build_resize_blur_operator():
    R = np.zeros((IMG, FMAP), np.float64)
    scale = FMAP / IMG
    for i in range(IMG):
        c = (i + 0.5) * scale - 0.5
        lo = int(np.floor(c))
        w = c - lo
        for j, wt in ((lo, 1.0 - w), (lo + 1, w)):
            R[i, min(max(j, 0), FMAP - 1)] += wt
    sigma = 4.0
    rad = int(3.0 * sigma + 0.5)
    x = np.arange(-rad, rad + 1, dtype=np.float64)
    k = np.exp(-0.5 * (x / sigma) ** 2)
    k /= k.sum()
    B = np.zeros((IMG, IMG), np.float64)
    for i in range(IMG):
        for t in range(2 * rad + 1):
            B[i, min(max(i + t - rad, 0), IMG - 1)] += k[t]
    return (B @ R).astype(np.float32)


_A_OP = _build_resize_blur_operator()


def _dotT(a, b, precision):
    return jax.lax.dot_general(
        a, b, (((1,), (1,)), ((), ())),
        precision=precision, preferred_element_type=jnp.float32)


def _knn_body(patch_ref, lib_ref, minv_ref, idx_ref, b2_ref, libb_ref,
              mtest_ref, sidx_ref, jstar_ref, sstar_ref, pm2_ref):
    kblk = pl.program_id(0)
    p = patch_ref[...]

    @pl.when(kblk == 0)
    def _():
        pm2_ref[...] = p * -2.0

    pm2 = pm2_ref[...]
    lb = lib_ref[...]
    ab2 = _dotT(lb, pm2, None)
    b2 = jnp.sum(lb * lb, axis=1, keepdims=True)
    b2_ref[...] = jnp.swapaxes(b2, 0, 1).reshape(1, 1, BK)
    libb_ref[...] = lb.astype(jnp.bfloat16)
    score = b2 + ab2
    bm = jnp.min(score, axis=0, keepdims=True)
    rows = jax.lax.broadcasted_iota(jnp.int32, (BK, Q), 0)
    ba = jnp.min(jnp.where(score == bm, rows, _INT_MAX),
                 axis=0, keepdims=True) + kblk * BK

    @pl.when(kblk == 0)
    def _():
        minv_ref[...] = bm
        idx_ref[...] = ba

    @pl.when(kblk > 0)
    def _():
        prev = minv_ref[...]
        better = bm < prev
        minv_ref[...] = jnp.where(better, bm, prev)
        idx_ref[...] = jnp.where(better, ba, idx_ref[...])

    @pl.when(kblk == NB - 1)
    def _():
        a2 = jnp.swapaxes(
            jnp.sum(p * p, axis=1, keepdims=True), 0, 1)
        mv = jnp.sqrt(jnp.maximum(minv_ref[...] + a2, 1e-12))
        minv_ref[...] = mv
        s_star = jnp.max(mv)
        lane = jax.lax.broadcasted_iota(jnp.int32, (1, Q), 1)
        s_idx = jnp.min(jnp.where(mv == s_star, lane, _INT_MAX))
        j_star = jnp.sum(jnp.where(lane == s_idx, idx_ref[...], 0))
        qrow = jax.lax.broadcasted_iota(jnp.int32, (Q, DF), 0)
        mtest_ref[...] = jnp.sum(jnp.where(qrow == s_idx, p, 0.0),
                                 axis=0, keepdims=True)
        sstar_ref[...] = jnp.full((1, 1), s_star, jnp.float32)
        sidx_ref[...] = jnp.full((1, 1), s_idx, jnp.int32)
        jstar_ref[...] = jnp.full((1, 1), j_star, jnp.int32)


def _reweight_body(jstar_ref, lib_ref, b2_ref, libf_ref, mtest_ref,
                   sstar_ref, m26_ref, a_ref, s_ref, smap_ref,
                   wd2_ref, td2_ref, ms_ref, sem):
    kblk = pl.program_id(0)

    @pl.when(kblk == 0)
    def _():
        j = jstar_ref[0, 0]
        pltpu.make_async_copy(
            libf_ref.at[pl.ds(j, 1), :], ms_ref, sem).start()
        pltpu.make_async_copy(
            libf_ref.at[pl.ds(j, 1), :], ms_ref, sem).wait()

    lb = lib_ref[...]
    b2 = b2_ref[0]
    ms = ms_ref[...]
    mt = mtest_ref[...]
    mm = jnp.concatenate([ms, mt], axis=0).astype(jnp.bfloat16)
    pair = _dotT(mm, lb, None)
    msq = jnp.sum(ms * ms)
    tsq = jnp.sum(mt * mt)
    wd2_ref[pl.ds(kblk, 1), :] = b2 - 2.0 * pair[0:1, :] + msq
    td2_ref[pl.ds(kblk, 1), :] = b2 - 2.0 * pair[1:2, :] + tsq

    @pl.when(kblk == 0)
    def _():
        a = a_ref[...]
        m = m26_ref[...]
        am = jax.lax.dot_general(
            a, m, (((1,), (0,)), ((), ())),
            precision=jax.lax.Precision.HIGHEST,
            preferred_element_type=jnp.float32)
        smap_ref[...] = _dotT(am, a, jax.lax.Precision.HIGHEST)

    @pl.when(kblk == NB2 - 1)
    def _():
        wd2 = wd2_ref[...]
        td2 = td2_ref[...]
        lane = (jax.lax.broadcasted_iota(jnp.int32, (NB2, BK2), 0) * BK2 +
                jax.lax.broadcasted_iota(jnp.int32, (NB2, BK2), 1))
        big = jnp.float32(3.0e38)

        def first_argmin(w):
            return jnp.min(jnp.where(w == jnp.min(w), lane, _INT_MAX))

        i1 = first_argmin(wd2)
        w2 = jnp.where(lane == i1, big, wd2)
        i2 = first_argmin(w2)
        w3 = jnp.where(lane == i2, big, w2)
        i3 = first_argmin(w3)
        t2 = jnp.sqrt(jnp.maximum(
            jnp.sum(jnp.where(lane == i2, td2, 0.0)), 0.0))
        t3 = jnp.sqrt(jnp.maximum(
            jnp.sum(jnp.where(lane == i3, td2, 0.0)), 0.0))
        dsq = jnp.sqrt(jnp.float32(DF))
        s_star = sstar_ref[0, 0]
        w = 1.0 - jnp.exp(s_star / dsq) / (jnp.exp(t2 / dsq) +
                                           jnp.exp(t3 / dsq))
        s_ref[...] = jnp.full((1, 1), w * s_star, jnp.float32)


def kernel(patch, patch_lib):
    minv, idx, b2, libb, mtest, sidx, jstar, sstar = pl.pallas_call(
        _knn_body,
        grid=(NB,),
        in_specs=[
            pl.BlockSpec((Q, DF), lambda k: (0, 0)),
            pl.BlockSpec((BK, DF), lambda k: (k, 0)),
        ],
        out_specs=[
            pl.BlockSpec((1, Q), lambda k: (0, 0)),
            pl.BlockSpec((1, Q), lambda k: (0, 0)),
            pl.BlockSpec((1, 1, BK), lambda k: (k // (BK2 // BK), 0,
                                                 k % (BK2 // BK))),
            pl.BlockSpec((BK, DF), lambda k: (k, 0)),
            pl.BlockSpec((1, DF), lambda k: (0, 0)),
            pl.BlockSpec((1, 1), lambda k: (0, 0)),
            pl.BlockSpec((1, 1), lambda k: (0, 0)),
            pl.BlockSpec((1, 1), lambda k: (0, 0)),
        ],
        scratch_shapes=[
            pltpu.VMEM((Q, DF), jnp.float32),
        ],
        out_shape=[
            jax.ShapeDtypeStruct((1, Q), jnp.float32),
            jax.ShapeDtypeStruct((1, Q), jnp.int32),
            jax.ShapeDtypeStruct((NB2, 1, BK2), jnp.float32),
            jax.ShapeDtypeStruct((KLIB, DF), jnp.bfloat16),
            jax.ShapeDtypeStruct((1, DF), jnp.float32),
            jax.ShapeDtypeStruct((1, 1), jnp.int32),
            jax.ShapeDtypeStruct((1, 1), jnp.int32),
            jax.ShapeDtypeStruct((1, 1), jnp.float32),
        ],
    )(patch, patch_lib)
    del idx, sidx

    m26 = minv.reshape(FMAP, FMAP)
    b2r = b2

    s_out, smap = pl.pallas_call(
        _reweight_body,
        grid=(NB2,),
        in_specs=[
            pl.BlockSpec(memory_space=pltpu.MemorySpace.SMEM),
            pl.BlockSpec((BK2, DF), lambda k: (k, 0)),
            pl.BlockSpec((1, 1, BK2), lambda k: (k, 0, 0)),
            pl.BlockSpec(memory_space=pltpu.MemorySpace.HBM),
            pl.BlockSpec((1, DF), lambda k: (0, 0)),
            pl.BlockSpec((1, 1), lambda k: (0, 0)),
            pl.BlockSpec((FMAP, FMAP), lambda k: (0, 0)),
            pl.BlockSpec((IMG, FMAP), lambda k: (0, 0)),
        ],
        out_specs=[
            pl.BlockSpec((1, 1), lambda k: (0, 0)),
            pl.BlockSpec((IMG, IMG), lambda k: (0, 0)),
        ],
        scratch_shapes=[
            pltpu.VMEM((NB2, BK2), jnp.float32),
            pltpu.VMEM((NB2, BK2), jnp.float32),
            pltpu.VMEM((1, DF), jnp.float32),
            pltpu.SemaphoreType.DMA,
        ],
        out_shape=[
            jax.ShapeDtypeStruct((1, 1), jnp.float32),
            jax.ShapeDtypeStruct((IMG, IMG), jnp.float32),
        ],
    )(jstar, libb, b2r, patch_lib, mtest, sstar, m26, jnp.asarray(_A_OP))

    return (s_out[0, 0], smap.reshape(1, 1, IMG, IMG))


# speedup vs baseline: 1.0079x; 1.0079x over previous
import numpy as np
import jax
import jax.numpy as jnp
from jax.experimental import pallas as pl
from jax.experimental.pallas import tpu as pltpu

FMAP = 26
IMG = 224
DF = 1536
KLIB = 16384
Q = FMAP * FMAP

BK = 2048
NB = KLIB // BK
BK2 = 4096
NB2 = KLIB // BK2

_INT_MAX = np.int32(2**31 - 1)


def _build_resize_blur_operator():
    R = np.zeros((IMG, FMAP), np.float64)
    scale = FMAP / IMG
    for i in range(IMG):
        c = (i + 0.5) * scale - 0.5
        lo = int(np.floor(c))
        w = c - lo
        for j, wt in ((lo, 1.0 - w), (lo + 1, w)):
            R[i, min(max(j, 0), FMAP - 1)] += wt
    sigma = 4.0
    rad = int(3.0 * sigma + 0.5)
    x = np.arange(-rad, rad + 1, dtype=np.float64)
    k = np.exp(-0.5 * (x / sigma) ** 2)
    k /= k.sum()
    B = np.zeros((IMG, IMG), np.float64)
    for i in range(IMG):
        for t in range(2 * rad + 1):
            B[i, min(max(i + t - rad, 0), IMG - 1)] += k[t]
    return (B @ R).astype(np.float32)


_A_OP = _build_resize_blur_operator()


def _dotT(a, b, precision):
    return jax.lax.dot_general(
        a, b, (((1,), (1,)), ((), ())),
        precision=precision, preferred_element_type=jnp.float32)


def _knn_body(patch_ref, lib_ref, minv_ref, idx_ref, b2_ref, libb_ref,
              mtest_ref, sidx_ref, jstar_ref, sstar_ref, pm2_ref):
    kblk = pl.program_id(0)
    p = patch_ref[...]

    @pl.when(kblk == 0)
    def _():
        pm2_ref[...] = p * -2.0

    pm2 = pm2_ref[...]
    lb = lib_ref[...]
    ab2 = _dotT(lb, pm2, None)
    b2 = jnp.sum(lb * lb, axis=1, keepdims=True)
    b2_ref[...] = jnp.swapaxes(b2, 0, 1).reshape(1, 1, BK)
    libb_ref[...] = lb.astype(jnp.bfloat16)
    score = b2 + ab2
    bm = jnp.min(score, axis=0, keepdims=True)
    rows = jax.lax.broadcasted_iota(jnp.int32, (BK, Q), 0)
    ba = jnp.min(jnp.where(score == bm, rows, _INT_MAX),
                 axis=0, keepdims=True) + kblk * BK

    @pl.when(kblk == 0)
    def _():
        minv_ref[...] = bm
        idx_ref[...] = ba

    @pl.when(kblk > 0)
    def _():
        prev = minv_ref[...]
        better = bm < prev
        minv_ref[...] = jnp.where(better, bm, prev)
        idx_ref[...] = jnp.where(better, ba, idx_ref[...])

    @pl.when(kblk == NB - 1)
    def _():
        a2 = jnp.swapaxes(
            jnp.sum(p * p, axis=1, keepdims=True), 0, 1)
        mv = jnp.sqrt(jnp.maximum(minv_ref[...] + a2, 1e-12))
        minv_ref[...] = mv
        s_star = jnp.max(mv)
        lane = jax.lax.broadcasted_iota(jnp.int32, (1, Q), 1)
        s_idx = jnp.min(jnp.where(mv == s_star, lane, _INT_MAX))
        j_star = jnp.sum(jnp.where(lane == s_idx, idx_ref[...], 0))
        qrow = jax.lax.broadcasted_iota(jnp.int32, (Q, DF), 0)
        mtest_ref[...] = jnp.sum(jnp.where(qrow == s_idx, p, 0.0),
                                 axis=0, keepdims=True)
        sstar_ref[...] = jnp.full((1, 1), s_star, jnp.float32)
        sidx_ref[...] = jnp.full((1, 1), s_idx, jnp.int32)
        jstar_ref[...] = jnp.full((1, 1), j_star, jnp.int32)


def _reweight_body(lib_ref, b2_ref, mstar_ref, mtest_ref,
                   sstar_ref, m26_ref, a_ref, s_ref, smap_ref,
                   wd2_ref, td2_ref):
    kblk = pl.program_id(0)
    lb = lib_ref[...]
    b2 = b2_ref[0]
    ms = mstar_ref[...]
    mt = mtest_ref[...]
    mm = jnp.concatenate([ms, mt], axis=0).astype(jnp.bfloat16)
    pair = _dotT(mm, lb, None)
    msq = jnp.sum(ms * ms)
    tsq = jnp.sum(mt * mt)
    wd2_ref[pl.ds(kblk, 1), :] = b2 - 2.0 * pair[0:1, :] + msq
    td2_ref[pl.ds(kblk, 1), :] = b2 - 2.0 * pair[1:2, :] + tsq

    @pl.when(kblk == 0)
    def _():
        a = a_ref[...]
        m = m26_ref[...]
        am = jax.lax.dot_general(
            a, m, (((1,), (0,)), ((), ())),
            precision=jax.lax.Precision.HIGHEST,
            preferred_element_type=jnp.float32)
        smap_ref[...] = _dotT(am, a, jax.lax.Precision.HIGHEST)

    @pl.when(kblk == NB2 - 1)
    def _():
        wd2 = wd2_ref[...]
        td2 = td2_ref[...]
        lane = (jax.lax.broadcasted_iota(jnp.int32, (NB2, BK2), 0) * BK2 +
                jax.lax.broadcasted_iota(jnp.int32, (NB2, BK2), 1))
        big = jnp.float32(3.0e38)

        def first_argmin(w):
            return jnp.min(jnp.where(w == jnp.min(w), lane, _INT_MAX))

        i1 = first_argmin(wd2)
        w2 = jnp.where(lane == i1, big, wd2)
        i2 = first_argmin(w2)
        w3 = jnp.where(lane == i2, big, w2)
        i3 = first_argmin(w3)
        t2 = jnp.sqrt(jnp.maximum(
            jnp.sum(jnp.where(lane == i2, td2, 0.0)), 0.0))
        t3 = jnp.sqrt(jnp.maximum(
            jnp.sum(jnp.where(lane == i3, td2, 0.0)), 0.0))
        dsq = jnp.sqrt(jnp.float32(DF))
        s_star = sstar_ref[0, 0]
        w = 1.0 - jnp.exp(s_star / dsq) / (jnp.exp(t2 / dsq) +
                                           jnp.exp(t3 / dsq))
        s_ref[...] = jnp.full((1, 1), w * s_star, jnp.float32)


def kernel(patch, patch_lib):
    minv, idx, b2, libb, mtest, sidx, jstar, sstar = pl.pallas_call(
        _knn_body,
        grid=(NB,),
        in_specs=[
            pl.BlockSpec((Q, DF), lambda k: (0, 0)),
            pl.BlockSpec((BK, DF), lambda k: (k, 0)),
        ],
        out_specs=[
            pl.BlockSpec((1, Q), lambda k: (0, 0)),
            pl.BlockSpec((1, Q), lambda k: (0, 0)),
            pl.BlockSpec((1, 1, BK), lambda k: (k // (BK2 // BK), 0,
                                                 k % (BK2 // BK))),
            pl.BlockSpec((BK, DF), lambda k: (k, 0)),
            pl.BlockSpec((1, DF), lambda k: (0, 0)),
            pl.BlockSpec((1, 1), lambda k: (0, 0)),
            pl.BlockSpec((1, 1), lambda k: (0, 0)),
            pl.BlockSpec((1, 1), lambda k: (0, 0)),
        ],
        scratch_shapes=[
            pltpu.VMEM((Q, DF), jnp.float32),
        ],
        out_shape=[
            jax.ShapeDtypeStruct((1, Q), jnp.float32),
            jax.ShapeDtypeStruct((1, Q), jnp.int32),
            jax.ShapeDtypeStruct((NB2, 1, BK2), jnp.float32),
            jax.ShapeDtypeStruct((KLIB, DF), jnp.bfloat16),
            jax.ShapeDtypeStruct((1, DF), jnp.float32),
            jax.ShapeDtypeStruct((1, 1), jnp.int32),
            jax.ShapeDtypeStruct((1, 1), jnp.int32),
            jax.ShapeDtypeStruct((1, 1), jnp.float32),
        ],
    )(patch, patch_lib)
    del idx, sidx

    mstar = jax.lax.dynamic_slice(
        patch_lib, (jstar[0, 0], jnp.int32(0)), (1, DF))
    m26 = minv.reshape(FMAP, FMAP)
    b2r = b2

    s_out, smap = pl.pallas_call(
        _reweight_body,
        grid=(NB2,),
        in_specs=[
            pl.BlockSpec((BK2, DF), lambda k: (k, 0)),
            pl.BlockSpec((1, 1, BK2), lambda k: (k, 0, 0)),
            pl.BlockSpec((1, DF), lambda k: (0, 0)),
            pl.BlockSpec((1, DF), lambda k: (0, 0)),
            pl.BlockSpec((1, 1), lambda k: (0, 0)),
            pl.BlockSpec((FMAP, FMAP), lambda k: (0, 0)),
            pl.BlockSpec((IMG, FMAP), lambda k: (0, 0)),
        ],
        out_specs=[
            pl.BlockSpec((1, 1), lambda k: (0, 0)),
            pl.BlockSpec((IMG, IMG), lambda k: (0, 0)),
        ],
        scratch_shapes=[
            pltpu.VMEM((NB2, BK2), jnp.float32),
            pltpu.VMEM((NB2, BK2), jnp.float32),
        ],
        out_shape=[
            jax.ShapeDtypeStruct((1, 1), jnp.float32),
            jax.ShapeDtypeStruct((IMG, IMG), jnp.float32),
        ],
    )(libb, b2r, mstar, mtest, sstar, m26, jnp.asarray(_A_OP))

    return (s_out[0, 0], smap.reshape(1, 1, IMG, IMG))
